# baseline (speedup 1.0000x reference)
import jax
import jax.numpy as jnp
from jax import lax
from jax.experimental import pallas as pl
from jax.experimental.pallas import tpu as pltpu
from jax.experimental.pallas import tpu_sc as plsc

N = 10000
NP = 10240
E = 320000
D = 128
DH = D // 2

NC = 2
NS = 16
NW = NC * NS
EPW = E // NW
EPWP = 10240
CHUNK = 128
NCHUNK = EPWP // CHUNK
RPT = NP // NS
ZROWS = 128
NB = 5
PD = 2
NG = NCHUNK // NB

_mesh = plsc.VectorSubcoreMesh(core_axis_name="c", subcore_axis_name="s")


def _deg_body(dst_hbm, out_hbm, dstv, onesv, zbuf, degsp):
    c = lax.axis_index("c")
    s = lax.axis_index("s")
    wid = s * NC + c
    pltpu.sync_copy(dst_hbm.at[wid], dstv)
    ones16 = jnp.ones((16,), jnp.float32)
    zeros16 = jnp.zeros((16,), jnp.float32)

    def fill(i, _):
        onesv[i, :] = ones16
        return 0

    lax.fori_loop(0, CHUNK, fill, 0)

    def zfill(i, _):
        zbuf[i, :] = zeros16
        return 0

    lax.fori_loop(0, ZROWS, zfill, 0)
    for r in range(RPT // ZROWS):
        pltpu.sync_copy(zbuf, degsp.at[pl.ds(s * RPT + r * ZROWS, ZROWS)])
    plsc.subcore_barrier()

    def body(j, _):
        pltpu.sync_copy(onesv, degsp.at[dstv.at[j]], add=True)
        return 0

    lax.fori_loop(0, NCHUNK, body, 0)
    plsc.subcore_barrier()
    for r in range(RPT // ZROWS):
        sl = pl.ds(s * RPT + r * ZROWS, ZROWS)
        pltpu.sync_copy(degsp.at[sl], out_hbm.at[c, sl])


_deg_call = pl.kernel(
    _deg_body,
    out_type=jax.ShapeDtypeStruct((NC, NP, 16), jnp.float32),
    mesh=_mesh,
    scratch_types=[
        pltpu.VMEM((NCHUNK, CHUNK), jnp.int32),
        pltpu.VMEM((CHUNK, 16), jnp.float32),
        pltpu.VMEM((ZROWS, 16), jnp.float32),
        pltpu.VMEM_SHARED((NP, 16), jnp.float32),
    ],
)


def _agg_body(glo_hbm, ghi_hbm, src_hbm, dst_hbm, olo_hbm, ohi_hbm,
              srcv, dstv, zbuf, rows, aggsp, gsem, ssem):
    c = lax.axis_index("c")
    s = lax.axis_index("s")
    wid = s * NC + c
    pltpu.sync_copy(src_hbm.at[wid], srcv)
    pltpu.sync_copy(dst_hbm.at[wid], dstv)
    zeros16 = jnp.zeros((16,), jnp.float32)

    def zfill(i, _):
        for j in range(DH // 16):
            zbuf[i, pl.ds(j * 16, 16)] = zeros16
        return 0

    lax.fori_loop(0, ZROWS, zfill, 0)

    for gh, outh in ((glo_hbm, olo_hbm), (ghi_hbm, ohi_hbm)):
        for r in range(RPT // ZROWS):
            pltpu.sync_copy(zbuf, aggsp.at[pl.ds(s * RPT + r * ZROWS, ZROWS)])
        plsc.subcore_barrier()

        def _wait_gather(b, j):
            pltpu.make_async_copy(
                gh.at[srcv.at[j]], rows.at[b], gsem.at[b]).wait()

        def _wait_scatter(b, j):
            pltpu.make_async_copy(
                rows.at[b], aggsp.at[dstv.at[j]], ssem.at[b]).wait()

        def _start_gather(b, j):
            pltpu.async_copy(gh.at[srcv.at[j]], rows.at[b], gsem.at[b])

        for b in range(PD):
            _start_gather(b, b)

        def group(g, _):
            for b in range(NB):
                j = g * NB + b
                bn = (b + PD) % NB
                _wait_gather(b, j)
                pltpu.async_copy(rows.at[b], aggsp.at[dstv.at[j]],
                                 ssem.at[b], add=True)
                if b < NB - PD:
                    @pl.when(g > 0)
                    def _():
                        _wait_scatter(bn, j + PD - NB)
                        _start_gather(bn, j + PD)

                    @pl.when(g == 0)
                    def _():
                        _start_gather(bn, j + PD)
                else:
                    _wait_scatter(bn, j + PD - NB)

                    @pl.when(g < NG - 1)
                    def _():
                        _start_gather(bn, j + PD)
            return 0

        lax.fori_loop(0, NG, group, 0)
        for b in range(PD, NB):
            _wait_scatter(b, NCHUNK - NB + b)
        plsc.subcore_barrier()
        for r in range(RPT // ZROWS):
            sl = pl.ds(s * RPT + r * ZROWS, ZROWS)
            pltpu.sync_copy(aggsp.at[sl], outh.at[c, sl])


_agg_call = pl.kernel(
    _agg_body,
    out_type=[
        jax.ShapeDtypeStruct((NC, NP, DH), jnp.float32),
        jax.ShapeDtypeStruct((NC, NP, DH), jnp.float32),
    ],
    mesh=_mesh,
    compiler_params=pltpu.CompilerParams(use_tc_tiling_on_sc=False),
    scratch_types=[
        pltpu.VMEM((NCHUNK, CHUNK), jnp.int32),
        pltpu.VMEM((NCHUNK, CHUNK), jnp.int32),
        pltpu.VMEM((ZROWS, DH), jnp.float32),
        pltpu.VMEM((NB, CHUNK, DH), jnp.float32),
        pltpu.VMEM_SHARED((NP, DH), jnp.float32),
        pltpu.SemaphoreType.DMA((NB,)),
        pltpu.SemaphoreType.DMA((NB,)),
    ],
)

ROWBLK = 1024
GRID = NP // ROWBLK


def _mm1_body(deg_ref, x_ref, w_ref, glo_ref, ghi_ref, dinv_ref):
    d = deg_ref[...]
    tot = d[0] + d[1] + 1.0
    dinv = lax.rsqrt(tot)
    dcol = lax.slice(dinv, (0, 0), (ROWBLK, 1))
    dinv128 = jnp.broadcast_to(dcol, (ROWBLK, D))
    h = jnp.dot(x_ref[...], w_ref[...], preferred_element_type=jnp.float32)
    g = h * dinv128
    glo_ref[...] = g[:, :DH]
    ghi_ref[...] = g[:, DH:]
    dinv_ref[...] = dinv128


def _mm2_body(plo_ref, phi_ref, glo_ref, ghi_ref, dinv_ref, w_ref, b_ref,
              g2lo_ref, g2hi_ref):
    a = jnp.concatenate(
        [plo_ref[0] + plo_ref[1], phi_ref[0] + phi_ref[1]], axis=1)
    g = jnp.concatenate([glo_ref[...], ghi_ref[...]], axis=1)
    z = (a + g) * dinv_ref[...] + b_ref[...]
    z = jnp.where(z > 0, z, 0.01 * z)
    h2 = jnp.dot(z, w_ref[...], preferred_element_type=jnp.float32)
    g2 = h2 * dinv_ref[...]
    g2lo_ref[...] = g2[:, :DH]
    g2hi_ref[...] = g2[:, DH:]


def _fin_body(plo_ref, phi_ref, g2lo_ref, g2hi_ref, dinv_ref, b_ref, out_ref):
    a = jnp.concatenate(
        [plo_ref[0] + plo_ref[1], phi_ref[0] + phi_ref[1]], axis=1)
    g2 = jnp.concatenate([g2lo_ref[...], g2hi_ref[...]], axis=1)
    out_ref[...] = (a + g2) * dinv_ref[...] + b_ref[...]


_row_spec = pl.BlockSpec((ROWBLK, D), lambda i: (i, 0))
_half_spec = pl.BlockSpec((ROWBLK, DH), lambda i: (i, 0))
_pair_spec = pl.BlockSpec((NC, ROWBLK, DH), lambda i: (0, i, 0))
_w_spec = pl.BlockSpec((D, D), lambda i: (0, 0))
_b_spec = pl.BlockSpec((1, D), lambda i: (0, 0))

_mm1_call = pl.pallas_call(
    _mm1_body,
    grid=(GRID,),
    in_specs=[
        pl.BlockSpec((NC, ROWBLK, 16), lambda i: (0, i, 0)),
        _row_spec,
        _w_spec,
    ],
    out_specs=[_half_spec, _half_spec, _row_spec],
    out_shape=[
        jax.ShapeDtypeStruct((NP, DH), jnp.float32),
        jax.ShapeDtypeStruct((NP, DH), jnp.float32),
        jax.ShapeDtypeStruct((NP, D), jnp.float32),
    ],
)

_mm2_call = pl.pallas_call(
    _mm2_body,
    grid=(GRID,),
    in_specs=[_pair_spec, _pair_spec, _half_spec, _half_spec, _row_spec,
              _w_spec, _b_spec],
    out_specs=[_half_spec, _half_spec],
    out_shape=[
        jax.ShapeDtypeStruct((NP, DH), jnp.float32),
        jax.ShapeDtypeStruct((NP, DH), jnp.float32),
    ],
)

_fin_call = pl.pallas_call(
    _fin_body,
    grid=(GRID,),
    in_specs=[_pair_spec, _pair_spec, _half_spec, _half_spec, _row_spec,
              _b_spec],
    out_specs=_row_spec,
    out_shape=jax.ShapeDtypeStruct((NP, D), jnp.float32),
)


@jax.jit
def kernel(x, edge_index, batch, W1, b1, W2, b2):
    del batch
    ei = edge_index.reshape(2, NW, EPW)
    padv = N + (jnp.arange(EPWP - EPW, dtype=jnp.int32) % (NP - N))
    padv = jnp.broadcast_to(padv, (2, NW, EPWP - EPW))
    ei = jnp.concatenate([ei, padv], axis=2)
    src = ei[0].reshape(NW, NCHUNK, CHUNK)
    dst = ei[1].reshape(NW, NCHUNK, CHUNK)
    xp = jnp.pad(x, ((0, NP - N), (0, 0)))
    degp = _deg_call(dst)
    g1lo, g1hi, dinv = _mm1_call(degp, xp, W1)
    p1lo, p1hi = _agg_call(g1lo, g1hi, src, dst)
    g2lo, g2hi = _mm2_call(p1lo, p1hi, g1lo, g1hi, dinv, W2, b1.reshape(1, D))
    p2lo, p2hi = _agg_call(g2lo, g2hi, src, dst)
    out = _fin_call(p2lo, p2hi, g2lo, g2hi, dinv, b2.reshape(1, D))
    return out[:N]


# speedup vs baseline: 1.1290x; 1.1290x over previous
import jax
import jax.numpy as jnp
from jax import lax
from jax.experimental import pallas as pl
from jax.experimental.pallas import tpu as pltpu
from jax.experimental.pallas import tpu_sc as plsc

N = 10000
NP = 10240
E = 320000
D = 128
DH = D // 2

NC = 2
NS = 16
NW = NC * NS
EPW = E // NW
EPWP = 10240
CHUNK = 128
NCHUNK = EPWP // CHUNK
RPT = NP // NS
ZROWS = 128
NB = 5
PD = 3
NG = NCHUNK // NB

_mesh = plsc.VectorSubcoreMesh(core_axis_name="c", subcore_axis_name="s")


def _deg_body(dst_hbm, out_hbm, dstv, onesv, zbuf, degsp, dsem):
    c = lax.axis_index("c")
    s = lax.axis_index("s")
    wid = s * NC + c
    pltpu.sync_copy(dst_hbm.at[wid], dstv)
    ones16 = jnp.ones((16,), jnp.float32)
    zeros16 = jnp.zeros((16,), jnp.float32)

    def fill(i, _):
        onesv[i, :] = ones16
        return 0

    lax.fori_loop(0, CHUNK, fill, 0)

    def zfill(i, _):
        zbuf[i, :] = zeros16
        return 0

    lax.fori_loop(0, ZROWS, zfill, 0)
    for r in range(RPT // ZROWS):
        pltpu.sync_copy(zbuf, degsp.at[pl.ds(s * RPT + r * ZROWS, ZROWS)])
    plsc.subcore_barrier()

    def body(j, _):
        pltpu.async_copy(onesv, degsp.at[dstv.at[j]], dsem, add=True)
        return 0

    lax.fori_loop(0, NCHUNK, body, 0)

    def drain(j, _):
        pltpu.make_async_copy(onesv, degsp.at[dstv.at[j]], dsem).wait()
        return 0

    lax.fori_loop(0, NCHUNK, drain, 0)
    plsc.subcore_barrier()
    for r in range(RPT // ZROWS):
        sl = pl.ds(s * RPT + r * ZROWS, ZROWS)
        pltpu.sync_copy(degsp.at[sl], out_hbm.at[c, sl])


_deg_call = pl.kernel(
    _deg_body,
    out_type=jax.ShapeDtypeStruct((NC, NP, 16), jnp.float32),
    mesh=_mesh,
    scratch_types=[
        pltpu.VMEM((NCHUNK, CHUNK), jnp.int32),
        pltpu.VMEM((CHUNK, 16), jnp.float32),
        pltpu.VMEM((ZROWS, 16), jnp.float32),
        pltpu.VMEM_SHARED((NP, 16), jnp.float32),
        pltpu.SemaphoreType.DMA,
    ],
)


def _agg_body(glo_hbm, ghi_hbm, src_hbm, dst_hbm, olo_hbm, ohi_hbm,
              srcv, dstv, zbuf, rows, aggsp, gsem, ssem):
    c = lax.axis_index("c")
    s = lax.axis_index("s")
    wid = s * NC + c
    pltpu.sync_copy(src_hbm.at[wid], srcv)
    pltpu.sync_copy(dst_hbm.at[wid], dstv)
    zeros16 = jnp.zeros((16,), jnp.float32)

    def zfill(i, _):
        for j in range(DH // 16):
            zbuf[i, pl.ds(j * 16, 16)] = zeros16
        return 0

    lax.fori_loop(0, ZROWS, zfill, 0)

    for gh, outh in ((glo_hbm, olo_hbm), (ghi_hbm, ohi_hbm)):
        for r in range(RPT // ZROWS):
            pltpu.sync_copy(zbuf, aggsp.at[pl.ds(s * RPT + r * ZROWS, ZROWS)])
        plsc.subcore_barrier()

        def _wait_gather(b, j):
            pltpu.make_async_copy(
                gh.at[srcv.at[j]], rows.at[b], gsem.at[b]).wait()

        def _wait_scatter(b, j):
            pltpu.make_async_copy(
                rows.at[b], aggsp.at[dstv.at[j]], ssem.at[b]).wait()

        def _start_gather(b, j):
            pltpu.async_copy(gh.at[srcv.at[j]], rows.at[b], gsem.at[b])

        for b in range(PD):
            _start_gather(b, b)

        def group(g, _):
            for b in range(NB):
                j = g * NB + b
                bn = (b + PD) % NB
                _wait_gather(b, j)
                pltpu.async_copy(rows.at[b], aggsp.at[dstv.at[j]],
                                 ssem.at[b], add=True)
                if b < NB - PD:
                    @pl.when(g > 0)
                    def _():
                        _wait_scatter(bn, j + PD - NB)
                        _start_gather(bn, j + PD)

                    @pl.when(g == 0)
                    def _():
                        _start_gather(bn, j + PD)
                else:
                    _wait_scatter(bn, j + PD - NB)

                    @pl.when(g < NG - 1)
                    def _():
                        _start_gather(bn, j + PD)
            return 0

        lax.fori_loop(0, NG, group, 0)
        for b in range(PD, NB):
            _wait_scatter(b, NCHUNK - NB + b)
        plsc.subcore_barrier()
        for r in range(RPT // ZROWS):
            sl = pl.ds(s * RPT + r * ZROWS, ZROWS)
            pltpu.sync_copy(aggsp.at[sl], outh.at[c, sl])


_agg_call = pl.kernel(
    _agg_body,
    out_type=[
        jax.ShapeDtypeStruct((NC, NP, DH), jnp.float32),
        jax.ShapeDtypeStruct((NC, NP, DH), jnp.float32),
    ],
    mesh=_mesh,
    compiler_params=pltpu.CompilerParams(use_tc_tiling_on_sc=False),
    scratch_types=[
        pltpu.VMEM((NCHUNK, CHUNK), jnp.int32),
        pltpu.VMEM((NCHUNK, CHUNK), jnp.int32),
        pltpu.VMEM((ZROWS, DH), jnp.float32),
        pltpu.VMEM((NB, CHUNK, DH), jnp.float32),
        pltpu.VMEM_SHARED((NP, DH), jnp.float32),
        pltpu.SemaphoreType.DMA((NB,)),
        pltpu.SemaphoreType.DMA((NB,)),
    ],
)

ROWBLK = 1024
GRID = NP // ROWBLK


def _mm1_body(deg_ref, x_ref, w_ref, glo_ref, ghi_ref, dinv_ref):
    d = deg_ref[...]
    tot = d[0] + d[1] + 1.0
    dinv = lax.rsqrt(tot)
    dcol = lax.slice(dinv, (0, 0), (ROWBLK, 1))
    dinv128 = jnp.broadcast_to(dcol, (ROWBLK, D))
    h = jnp.dot(x_ref[...], w_ref[...], preferred_element_type=jnp.float32)
    g = h * dinv128
    glo_ref[...] = g[:, :DH]
    ghi_ref[...] = g[:, DH:]
    dinv_ref[...] = dinv128


def _mm2_body(plo_ref, phi_ref, glo_ref, ghi_ref, dinv_ref, w_ref, b_ref,
              g2lo_ref, g2hi_ref):
    a = jnp.concatenate(
        [plo_ref[0] + plo_ref[1], phi_ref[0] + phi_ref[1]], axis=1)
    g = jnp.concatenate([glo_ref[...], ghi_ref[...]], axis=1)
    z = (a + g) * dinv_ref[...] + b_ref[...]
    z = jnp.where(z > 0, z, 0.01 * z)
    h2 = jnp.dot(z, w_ref[...], preferred_element_type=jnp.float32)
    g2 = h2 * dinv_ref[...]
    g2lo_ref[...] = g2[:, :DH]
    g2hi_ref[...] = g2[:, DH:]


def _fin_body(plo_ref, phi_ref, g2lo_ref, g2hi_ref, dinv_ref, b_ref, out_ref):
    a = jnp.concatenate(
        [plo_ref[0] + plo_ref[1], phi_ref[0] + phi_ref[1]], axis=1)
    g2 = jnp.concatenate([g2lo_ref[...], g2hi_ref[...]], axis=1)
    out_ref[...] = (a + g2) * dinv_ref[...] + b_ref[...]


_row_spec = pl.BlockSpec((ROWBLK, D), lambda i: (i, 0))
_half_spec = pl.BlockSpec((ROWBLK, DH), lambda i: (i, 0))
_pair_spec = pl.BlockSpec((NC, ROWBLK, DH), lambda i: (0, i, 0))
_w_spec = pl.BlockSpec((D, D), lambda i: (0, 0))
_b_spec = pl.BlockSpec((1, D), lambda i: (0, 0))

_mm1_call = pl.pallas_call(
    _mm1_body,
    grid=(GRID,),
    in_specs=[
        pl.BlockSpec((NC, ROWBLK, 16), lambda i: (0, i, 0)),
        _row_spec,
        _w_spec,
    ],
    out_specs=[_half_spec, _half_spec, _row_spec],
    out_shape=[
        jax.ShapeDtypeStruct((NP, DH), jnp.float32),
        jax.ShapeDtypeStruct((NP, DH), jnp.float32),
        jax.ShapeDtypeStruct((NP, D), jnp.float32),
    ],
)

_mm2_call = pl.pallas_call(
    _mm2_body,
    grid=(GRID,),
    in_specs=[_pair_spec, _pair_spec, _half_spec, _half_spec, _row_spec,
              _w_spec, _b_spec],
    out_specs=[_half_spec, _half_spec],
    out_shape=[
        jax.ShapeDtypeStruct((NP, DH), jnp.float32),
        jax.ShapeDtypeStruct((NP, DH), jnp.float32),
    ],
)

_fin_call = pl.pallas_call(
    _fin_body,
    grid=(GRID,),
    in_specs=[_pair_spec, _pair_spec, _half_spec, _half_spec, _row_spec,
              _b_spec],
    out_specs=_row_spec,
    out_shape=jax.ShapeDtypeStruct((NP, D), jnp.float32),
)


@jax.jit
def kernel(x, edge_index, batch, W1, b1, W2, b2):
    del batch
    ei = edge_index.reshape(2, NW, EPW)
    padv = N + (jnp.arange(EPWP - EPW, dtype=jnp.int32) % (NP - N))
    padv = jnp.broadcast_to(padv, (2, NW, EPWP - EPW))
    ei = jnp.concatenate([ei, padv], axis=2)
    src = ei[0].reshape(NW, NCHUNK, CHUNK)
    dst = ei[1].reshape(NW, NCHUNK, CHUNK)
    xp = jnp.pad(x, ((0, NP - N), (0, 0)))
    degp = _deg_call(dst)
    g1lo, g1hi, dinv = _mm1_call(degp, xp, W1)
    p1lo, p1hi = _agg_call(g1lo, g1hi, src, dst)
    g2lo, g2hi = _mm2_call(p1lo, p1hi, g1lo, g1hi, dinv, W2, b1.reshape(1, D))
    p2lo, p2hi = _agg_call(g2lo, g2hi, src, dst)
    out = _fin_call(p2lo, p2hi, g2lo, g2hi, dinv, b2.reshape(1, D))
    return out[:N]
